# baseline (speedup 1.0000x reference)
import functools

import jax
import jax.numpy as jnp
from jax import lax
from jax.experimental import pallas as pl
from jax.experimental.pallas import tpu as pltpu
from jax.experimental.pallas import tpu_sc as plsc

D = 64
DP = 128
SEQ = 200
B = 1024

NC = 2
NS = 16
NW = NC * NS
SEQS_PER_W = B // NW
NBUF = 4
NGROUPS = SEQS_PER_W // NBUF
GI = 40
NG = SEQ // GI


def _emb_body(x_hbm, tok_hbm, pos_hbm, out_hbm, idx_all, rows_v, pos_sh,
              sem_pre, sem_g, sem_wb):
    wid = lax.axis_index("s") * NC + lax.axis_index("c")

    @pl.when(lax.axis_index("s") == 0)
    def _():
        pltpu.sync_copy(pos_hbm.at[pl.ds(0, SEQ), pl.ds(0, D)], pos_sh)

    plsc.subcore_barrier()

    pltpu.sync_copy(x_hbm.at[pl.ds(wid * SEQS_PER_W * SEQ, SEQS_PER_W * SEQ)],
                    idx_all)

    seq0 = wid * SEQS_PER_W

    def group(g, carry):
        for b in range(NBUF):
            j = g * NBUF + b

            @pl.when(g > 0)
            def _(b=b, j=j):
                pltpu.make_async_copy(
                    rows_v.at[b], out_hbm.at[seq0 + j - NBUF], sem_wb.at[b]
                ).wait()

            pltpu.async_copy(pos_sh, rows_v.at[b, :, pl.ds(0, D)],
                             sem_pre.at[b])

        for b in range(NBUF):
            j = g * NBUF + b
            pltpu.make_async_copy(pos_sh, rows_v.at[b, :, pl.ds(0, D)],
                                  sem_pre.at[b]).wait()
            for h in range(NG):
                pltpu.async_copy(
                    tok_hbm.at[idx_all.at[pl.ds(j * SEQ + h * GI, GI)]],
                    rows_v.at[b, pl.ds(h * GI, GI)],
                    sem_g.at[b], add=True,
                )

        for b in range(NBUF):
            j = g * NBUF + b
            for h in range(NG):
                pltpu.make_async_copy(
                    tok_hbm.at[idx_all.at[pl.ds(j * SEQ + h * GI, GI)]],
                    rows_v.at[b, pl.ds(h * GI, GI)],
                    sem_g.at[b],
                ).wait()
            pltpu.async_copy(rows_v.at[b], out_hbm.at[seq0 + j], sem_wb.at[b])
        return carry

    lax.fori_loop(0, NGROUPS, group, 0)

    for b in range(NBUF):
        j = (NGROUPS - 1) * NBUF + b
        pltpu.make_async_copy(
            rows_v.at[b], out_hbm.at[seq0 + j], sem_wb.at[b]
        ).wait()


_BX = 8192


def _transpose_pad_body(tokT_ref, out_ref):
    blk = tokT_ref[...]
    out_ref[:, :D] = blk.T
    out_ref[:, D:] = jnp.zeros((_BX, DP - D), jnp.float32)


def _transpose_pad(tokT):
    v = tokT.shape[1]
    grid = (v + _BX - 1) // _BX
    return pl.pallas_call(
        _transpose_pad_body,
        grid=(grid,),
        in_specs=[pl.BlockSpec((D, _BX), lambda i: (0, i))],
        out_specs=pl.BlockSpec((_BX, DP), lambda i: (i, 0)),
        out_shape=jax.ShapeDtypeStruct((v, DP), jnp.float32),
    )(tokT)


@jax.jit
def kernel(x, token_emb, pos_emb):
    b, l = x.shape
    x_flat = x.astype(jnp.int32).reshape(b * l)
    tok_pad = _transpose_pad(token_emb.T)
    pos_pad = jnp.pad(pos_emb, ((0, 0), (0, DP - D)))
    mesh = plsc.VectorSubcoreMesh(core_axis_name="c", subcore_axis_name="s")
    k = pl.kernel(
        _emb_body,
        out_type=jax.ShapeDtypeStruct((b, l, DP), jnp.float32),
        mesh=mesh,
        scratch_types=[
            pltpu.VMEM((SEQS_PER_W * SEQ,), jnp.int32),
            pltpu.VMEM((NBUF, SEQ, DP), jnp.float32),
            pltpu.VMEM_SHARED((SEQ, D), jnp.float32),
            pltpu.SemaphoreType.DMA((NBUF,)),
            pltpu.SemaphoreType.DMA((NBUF,)),
            pltpu.SemaphoreType.DMA((NBUF,)),
        ],
        compiler_params=pltpu.CompilerParams(use_tc_tiling_on_sc=False),
    )
    return k(x_flat, tok_pad, pos_pad)[:, :, :D]


# speedup vs baseline: 1.0355x; 1.0355x over previous
import functools

import jax
import jax.numpy as jnp
from jax import lax
from jax.experimental import pallas as pl
from jax.experimental.pallas import tpu as pltpu
from jax.experimental.pallas import tpu_sc as plsc

D = 64
DP = 128
SEQ = 200
B = 1024

NC = 2
NS = 16
NW = NC * NS
SEQS_PER_W = B // NW
NBUF = 4
NGROUPS = SEQS_PER_W // NBUF
GI = 40
NG = SEQ // GI


def _emb_body(x_hbm, tok_hbm, pos_hbm, out_hbm, idx_all, rows_v, pos_sh,
              sem_pre, sem_g, sem_wb):
    wid = lax.axis_index("s") * NC + lax.axis_index("c")

    @pl.when(lax.axis_index("s") == 0)
    def _():
        pltpu.sync_copy(pos_hbm, pos_sh)

    plsc.subcore_barrier()

    pltpu.sync_copy(x_hbm.at[pl.ds(wid * SEQS_PER_W * SEQ, SEQS_PER_W * SEQ)],
                    idx_all)

    seq0 = wid * SEQS_PER_W

    def group(g, carry):
        for b in range(NBUF):
            j = g * NBUF + b

            @pl.when(g > 0)
            def _(b=b, j=j):
                pltpu.make_async_copy(
                    rows_v.at[b], out_hbm.at[seq0 + j - NBUF], sem_wb.at[b]
                ).wait()

            pltpu.async_copy(pos_sh, rows_v.at[b], sem_pre.at[b])

        for b in range(NBUF):
            j = g * NBUF + b
            pltpu.make_async_copy(pos_sh, rows_v.at[b], sem_pre.at[b]).wait()
            for h in range(NG):
                pltpu.async_copy(
                    tok_hbm.at[idx_all.at[pl.ds(j * SEQ + h * GI, GI)]],
                    rows_v.at[b, pl.ds(h * GI, GI)],
                    sem_g.at[b], add=True,
                )

        for b in range(NBUF):
            j = g * NBUF + b
            for h in range(NG):
                pltpu.make_async_copy(
                    tok_hbm.at[idx_all.at[pl.ds(j * SEQ + h * GI, GI)]],
                    rows_v.at[b, pl.ds(h * GI, GI)],
                    sem_g.at[b],
                ).wait()
            pltpu.async_copy(rows_v.at[b], out_hbm.at[seq0 + j], sem_wb.at[b])
        return carry

    lax.fori_loop(0, NGROUPS, group, 0)

    for b in range(NBUF):
        j = (NGROUPS - 1) * NBUF + b
        pltpu.make_async_copy(
            rows_v.at[b], out_hbm.at[seq0 + j], sem_wb.at[b]
        ).wait()


_BX = 8192


def _transpose_pad_body(tokT_ref, out_ref):
    blk = tokT_ref[...]
    out_ref[:, :D] = blk.T
    out_ref[:, D:] = jnp.zeros((_BX, DP - D), jnp.float32)


def _transpose_pad(tokT):
    v = tokT.shape[1]
    grid = (v + _BX - 1) // _BX
    return pl.pallas_call(
        _transpose_pad_body,
        grid=(grid,),
        in_specs=[pl.BlockSpec((D, _BX), lambda i: (0, i))],
        out_specs=pl.BlockSpec((_BX, DP), lambda i: (i, 0)),
        out_shape=jax.ShapeDtypeStruct((v, DP), jnp.float32),
    )(tokT)


@jax.jit
def kernel(x, token_emb, pos_emb):
    b, l = x.shape
    x_flat = x.astype(jnp.int32).reshape(b * l)
    tok_pad = _transpose_pad(token_emb.T)
    pos_pad = jnp.pad(pos_emb, ((0, 0), (0, DP - D)))
    mesh = plsc.VectorSubcoreMesh(core_axis_name="c", subcore_axis_name="s")
    k = pl.kernel(
        _emb_body,
        out_type=jax.ShapeDtypeStruct((b, l, DP), jnp.float32),
        mesh=mesh,
        scratch_types=[
            pltpu.VMEM((SEQS_PER_W * SEQ,), jnp.int32),
            pltpu.VMEM((NBUF, SEQ, DP), jnp.float32),
            pltpu.VMEM_SHARED((SEQ, DP), jnp.float32),
            pltpu.SemaphoreType.DMA((NBUF,)),
            pltpu.SemaphoreType.DMA((NBUF,)),
            pltpu.SemaphoreType.DMA((NBUF,)),
        ],
        compiler_params=pltpu.CompilerParams(use_tc_tiling_on_sc=False),
    )
    return k(x_flat, tok_pad, pos_pad)[:, :, :D]


# speedup vs baseline: 1.0866x; 1.0494x over previous
import functools

import jax
import jax.numpy as jnp
from jax import lax
from jax.experimental import pallas as pl
from jax.experimental.pallas import tpu as pltpu
from jax.experimental.pallas import tpu_sc as plsc

D = 64
DP = 128
SEQ = 200
B = 1024

NC = 2
NS = 16
NW = NC * NS
SEQS_PER_W = B // NW
NBUF = 4
NGROUPS = SEQS_PER_W // NBUF
GI = 40
NG = SEQ // GI


def _emb_body(x_hbm, tok_hbm, pos_hbm, out_hbm, idx_all, rows_v, pos_sh,
              sem_pre, sem_g, sem_wb):
    wid = lax.axis_index("s") * NC + lax.axis_index("c")

    @pl.when(lax.axis_index("s") == 0)
    def _():
        pltpu.sync_copy(pos_hbm, pos_sh)

    plsc.subcore_barrier()

    pltpu.sync_copy(x_hbm.at[pl.ds(wid * SEQS_PER_W * SEQ, SEQS_PER_W * SEQ)],
                    idx_all)

    seq0 = wid * SEQS_PER_W

    def group(g, carry):
        for b in range(NBUF):
            j = g * NBUF + b

            @pl.when(g > 0)
            def _(b=b, j=j):
                pltpu.make_async_copy(
                    rows_v.at[b], out_hbm.at[seq0 + j - NBUF], sem_wb.at[b]
                ).wait()

            pltpu.async_copy(pos_sh, rows_v.at[b], sem_pre.at[b])

        for b in range(NBUF):
            j = g * NBUF + b
            pltpu.make_async_copy(pos_sh, rows_v.at[b], sem_pre.at[b]).wait()
            for h in range(NG):
                pltpu.async_copy(
                    tok_hbm.at[idx_all.at[pl.ds(j * SEQ + h * GI, GI)]],
                    rows_v.at[b, pl.ds(h * GI, GI)],
                    sem_g.at[b], add=True,
                )

        for b in range(NBUF):
            j = g * NBUF + b
            for h in range(NG):
                pltpu.make_async_copy(
                    tok_hbm.at[idx_all.at[pl.ds(j * SEQ + h * GI, GI)]],
                    rows_v.at[b, pl.ds(h * GI, GI)],
                    sem_g.at[b],
                ).wait()
            pltpu.async_copy(rows_v.at[b], out_hbm.at[seq0 + j], sem_wb.at[b])
        return carry

    lax.fori_loop(0, NGROUPS, group, 0)

    for b in range(NBUF):
        j = (NGROUPS - 1) * NBUF + b
        pltpu.make_async_copy(
            rows_v.at[b], out_hbm.at[seq0 + j], sem_wb.at[b]
        ).wait()


_BX = 16384


def _transpose_pad_body(tokT_ref, out_ref):
    blk = tokT_ref[...]
    out_ref[:, :D] = blk.T
    out_ref[:, D:] = jnp.zeros((_BX, DP - D), jnp.float32)


def _transpose_pad(tokT):
    v = tokT.shape[1]
    grid = (v + _BX - 1) // _BX
    return pl.pallas_call(
        _transpose_pad_body,
        grid=(grid,),
        in_specs=[pl.BlockSpec((D, _BX), lambda i: (0, i))],
        out_specs=pl.BlockSpec((_BX, DP), lambda i: (i, 0)),
        out_shape=jax.ShapeDtypeStruct((v, DP), jnp.float32),
    )(tokT)


@jax.jit
def kernel(x, token_emb, pos_emb):
    b, l = x.shape
    x_flat = x.astype(jnp.int32).reshape(b * l)
    tok_pad = _transpose_pad(token_emb.T)
    pos_pad = jnp.pad(pos_emb, ((0, 0), (0, DP - D)))
    mesh = plsc.VectorSubcoreMesh(core_axis_name="c", subcore_axis_name="s")
    k = pl.kernel(
        _emb_body,
        out_type=jax.ShapeDtypeStruct((b, l, DP), jnp.float32),
        mesh=mesh,
        scratch_types=[
            pltpu.VMEM((SEQS_PER_W * SEQ,), jnp.int32),
            pltpu.VMEM((NBUF, SEQ, DP), jnp.float32),
            pltpu.VMEM_SHARED((SEQ, DP), jnp.float32),
            pltpu.SemaphoreType.DMA((NBUF,)),
            pltpu.SemaphoreType.DMA((NBUF,)),
            pltpu.SemaphoreType.DMA((NBUF,)),
        ],
        compiler_params=pltpu.CompilerParams(use_tc_tiling_on_sc=False),
    )
    return k(x_flat, tok_pad, pos_pad)[:, :, :D]


# speedup vs baseline: 1.1032x; 1.0153x over previous
import functools

import jax
import jax.numpy as jnp
from jax import lax
from jax.experimental import pallas as pl
from jax.experimental.pallas import tpu as pltpu
from jax.experimental.pallas import tpu_sc as plsc

D = 64
DP = 128
SEQ = 200
B = 1024

NC = 2
NS = 16
NW = NC * NS
SEQS_PER_W = B // NW
NBUF = 4
NGROUPS = SEQS_PER_W // NBUF
GI = 40
NG = SEQ // GI


def _emb_body(x_hbm, tok_hbm, pos_hbm, out_hbm, idx_all, rows_v, pos_sh,
              sem_pre, sem_g, sem_wb):
    wid = lax.axis_index("s") * NC + lax.axis_index("c")

    @pl.when(lax.axis_index("s") == 0)
    def _():
        pltpu.sync_copy(pos_hbm, pos_sh)

    plsc.subcore_barrier()

    pltpu.sync_copy(x_hbm.at[pl.ds(wid * SEQS_PER_W * SEQ, SEQS_PER_W * SEQ)],
                    idx_all)

    seq0 = wid * SEQS_PER_W

    def group(g, carry):
        for b in range(NBUF):
            j = g * NBUF + b

            @pl.when(g > 0)
            def _(b=b, j=j):
                pltpu.make_async_copy(
                    rows_v.at[b], out_hbm.at[seq0 + j - NBUF], sem_wb.at[b]
                ).wait()

            pltpu.async_copy(pos_sh, rows_v.at[b], sem_pre.at[b])

        for b in range(NBUF):
            j = g * NBUF + b
            pltpu.make_async_copy(pos_sh, rows_v.at[b], sem_pre.at[b]).wait()
            for h in range(NG):
                pltpu.async_copy(
                    tok_hbm.at[idx_all.at[pl.ds(j * SEQ + h * GI, GI)]],
                    rows_v.at[b, pl.ds(h * GI, GI)],
                    sem_g.at[b], add=True,
                )

        for b in range(NBUF):
            j = g * NBUF + b
            for h in range(NG):
                pltpu.make_async_copy(
                    tok_hbm.at[idx_all.at[pl.ds(j * SEQ + h * GI, GI)]],
                    rows_v.at[b, pl.ds(h * GI, GI)],
                    sem_g.at[b],
                ).wait()
            pltpu.async_copy(rows_v.at[b], out_hbm.at[seq0 + j], sem_wb.at[b])
        return carry

    lax.fori_loop(0, NGROUPS, group, 0)

    for b in range(NBUF):
        j = (NGROUPS - 1) * NBUF + b
        pltpu.make_async_copy(
            rows_v.at[b], out_hbm.at[seq0 + j], sem_wb.at[b]
        ).wait()


_BX = 32768


def _transpose_pad_body(tokT_ref, out_ref):
    blk = tokT_ref[...]
    out_ref[:, :D] = blk.T
    out_ref[:, D:] = jnp.zeros((_BX, DP - D), jnp.float32)


def _transpose_pad(tokT):
    v = tokT.shape[1]
    grid = (v + _BX - 1) // _BX
    return pl.pallas_call(
        _transpose_pad_body,
        grid=(grid,),
        in_specs=[pl.BlockSpec((D, _BX), lambda i: (0, i))],
        out_specs=pl.BlockSpec((_BX, DP), lambda i: (i, 0)),
        out_shape=jax.ShapeDtypeStruct((v, DP), jnp.float32),
    )(tokT)


@jax.jit
def kernel(x, token_emb, pos_emb):
    b, l = x.shape
    x_flat = x.astype(jnp.int32).reshape(b * l)
    tok_pad = _transpose_pad(token_emb.T)
    pos_pad = jnp.pad(pos_emb, ((0, 0), (0, DP - D)))
    mesh = plsc.VectorSubcoreMesh(core_axis_name="c", subcore_axis_name="s")
    k = pl.kernel(
        _emb_body,
        out_type=jax.ShapeDtypeStruct((b, l, DP), jnp.float32),
        mesh=mesh,
        scratch_types=[
            pltpu.VMEM((SEQS_PER_W * SEQ,), jnp.int32),
            pltpu.VMEM((NBUF, SEQ, DP), jnp.float32),
            pltpu.VMEM_SHARED((SEQ, DP), jnp.float32),
            pltpu.SemaphoreType.DMA((NBUF,)),
            pltpu.SemaphoreType.DMA((NBUF,)),
            pltpu.SemaphoreType.DMA((NBUF,)),
        ],
        compiler_params=pltpu.CompilerParams(use_tc_tiling_on_sc=False),
    )
    return k(x_flat, tok_pad, pos_pad)[:, :, :D]
